# baseline (speedup 1.0000x reference)
import functools

import jax
import jax.numpy as jnp
import numpy as np
from jax import lax
from jax.experimental import pallas as pl
from jax.experimental.pallas import tpu as pltpu
from jax.experimental.pallas import tpu_sc as plsc

COORD_DIM = 3
IN_CH = 128
OUT_CH = 128
HIDDEN = 64
RADIUS = 0.12
N_NODES = 10000
N_LATENT = 1024
BATCH = 2

N_PAD = 10240
WORDS = N_PAD // 16
WORDS_REAL = N_NODES // 16
KMAX = 144
NQ = BATCH * N_LATENT
NWORKERS = 32
QPW = N_LATENT // 16


def _lift_body(x_ref, w_ref, b_ref, o_ref):
    o_ref[...] = (lax.dot_general(
        x_ref[...], w_ref[...], (((1,), (1,)), ((), ())),
        preferred_element_type=jnp.float32) + b_ref[...])


def _mask_body(q_ref, x_ref, b2_ref, s_ref, w_ref, cc_ref):
    qc = q_ref[...]
    x3 = x_ref[0]
    r2 = RADIUS * RADIUS
    acc = jnp.zeros((256, 2048), jnp.float32)
    for d in range(COORD_DIM):
        diff = qc[:, d:d + 1] - x3[d:d + 1, :]
        acc = acc + diff * diff
    maskb = (acc <= r2).astype(jnp.bfloat16)
    words = lax.dot_general(maskb, b2_ref[...], (((1,), (0,)), ((), ())),
                            preferred_element_type=jnp.float32)
    w_ref[0] = words.astype(jnp.int32)
    cc_ref[0] = lax.dot_general(maskb, s_ref[...], (((1,), (0,)), ((), ())),
                                preferred_element_type=jnp.float32)


def _prefix_body(cc_ref, tri_ref, p_ref, c_ref):
    cc = cc_ref[0]
    pre = lax.dot_general(cc.astype(jnp.bfloat16), tri_ref[...],
                          (((1,), (0,)), ((), ())),
                          preferred_element_type=jnp.float32)
    p_ref[0] = jnp.minimum(pre, float(KMAX)).astype(jnp.int32)
    c_ref[...] = jnp.sum(cc, axis=1).reshape(1, 1, 256)


def _sc_body(words_hbm, prefix_hbm, cx_hbm, cy_hbm, cz_hbm, ltab_hbm,
             ygx_hbm, ygy_hbm, ygz_hbm, lg_hbm,
             wbuf, pbuf, idxbuf, cxb, cyb, czb, yxb, yyb, yzb, lbuf, spl,
             sem0):
    cid = lax.axis_index("c")
    sid = lax.axis_index("s")
    iota = lax.broadcasted_iota(jnp.int32, (16,), 0)
    pltpu.sync_copy(cx_hbm.at[pl.ds(cid * N_NODES, N_NODES)], cxb)
    pltpu.sync_copy(cy_hbm.at[pl.ds(cid * N_NODES, N_NODES)], cyb)
    pltpu.sync_copy(cz_hbm.at[pl.ds(cid * N_NODES, N_NODES)], czb)
    @pl.when(sid == 0)
    def _():
        pltpu.sync_copy(ltab_hbm.at[pl.ds(cid * N_NODES, N_NODES)], spl)
    plsc.subcore_barrier()

    def per_q(i, carry):
        q = cid * N_LATENT + sid * QPW + i
        pltpu.sync_copy(words_hbm.at[q], wbuf)
        pltpu.sync_copy(prefix_hbm.at[q], pbuf)
        for t in range(KMAX // 16):
            idxbuf[pl.ds(t * 16, 16)] = jnp.zeros((16,), jnp.int32)

        def group(g, carry2):
            wv = wbuf[pl.ds(g * 16, 16)]
            pv = pbuf[pl.ds(g * 16, 16)]
            gbase = (g * 16) * 16
            for l in range(16):
                msk = ((jnp.broadcast_to(wv[l], (16,)) >> iota) & 1) > 0
                nidx = (gbase + l * 16) + iota
                plsc.store_compressed(idxbuf.at[pl.ds(pv[l], 16)], nidx,
                                      mask=msk)
            return carry2

        lax.fori_loop(0, WORDS // 16, group, jnp.int32(0))
        cnt = pbuf[pl.ds(624, 16)][1]
        nch = (cnt + 15) >> 4

        def gch(t, carry3):
            pltpu.async_copy(spl.at[idxbuf.at[pl.ds(t * 16, 16)]],
                             lbuf.at[pl.ds(t * 16, 16)], sem0).wait()
            return carry3

        lax.fori_loop(0, nch, gch, jnp.int32(0))
        for t in range(KMAX // 16):
            iv = idxbuf[pl.ds(t * 16, 16)]
            yxb[pl.ds(t * 16, 16)] = plsc.load_gather(cxb, [iv])
            yyb[pl.ds(t * 16, 16)] = plsc.load_gather(cyb, [iv])
            yzb[pl.ds(t * 16, 16)] = plsc.load_gather(czb, [iv])
        pltpu.sync_copy(lbuf, lg_hbm.at[q])
        pltpu.sync_copy(yxb, ygx_hbm.at[q])
        pltpu.sync_copy(yyb, ygy_hbm.at[q])
        pltpu.sync_copy(yzb, ygz_hbm.at[q])
        return carry

    lax.fori_loop(0, QPW, per_q, jnp.int32(0))


QT = 16


def _msg_body(yx_ref, yy_ref, yz_ref, l_ref, lat_ref, c_ref, w0y_ref,
              w0x_ref, b0_ref, w1_ref, b1_ref, o_ref):
    w0 = w0y_ref[...]
    a = (yx_ref[...][:, :, None] * w0[0][None, None, :]
         + yy_ref[...][:, :, None] * w0[1][None, None, :]
         + yz_ref[...][:, :, None] * w0[2][None, None, :])
    bq = lax.dot_general(lat_ref[...], w0x_ref[...], (((1,), (0,)), ((), ())),
                         preferred_element_type=jnp.float32) + b0_ref[...]
    pre = a + bq[:, None, :]
    g = jax.nn.gelu(pre)
    k = lax.dot_general(g.reshape(QT * KMAX, HIDDEN).astype(jnp.bfloat16),
                        w1_ref[...].astype(jnp.bfloat16),
                        (((1,), (0,)), ((), ())),
                        preferred_element_type=jnp.float32) + b1_ref[...]
    msg = (k * l_ref[...]).reshape(QT, KMAX, OUT_CH)
    cnt = c_ref[0, 0, :]
    io3 = lax.broadcasted_iota(jnp.int32, (QT, KMAX, OUT_CH), 1)
    valid = io3 < cnt.astype(jnp.int32)[:, None, None]
    s = jnp.sum(jnp.where(valid, msg, 0.0), axis=1)
    denom = jnp.maximum(cnt, 1.0)
    o_ref[...] = s / denom[:, None]


def kernel(x_coord, pndata, latent_tokens_coord, W_lift, b_lift, W0, b0, W1, b1):
    pnd_flat = pndata.reshape(BATCH * N_NODES, IN_CH)
    xT = jnp.transpose(x_coord, (0, 2, 1))
    xT = jnp.pad(xT, ((0, 0), (0, 0), (0, N_PAD - N_NODES)),
                 constant_values=1e4)
    cx = x_coord[..., 0].reshape(BATCH * N_NODES)
    cy = x_coord[..., 1].reshape(BATCH * N_NODES)
    cz = x_coord[..., 2].reshape(BATCH * N_NODES)
    b2_np = np.zeros((2048, 128), np.float32)
    for n in range(2048):
        b2_np[n, n // 16] = float(1 << (n % 16))
    b2 = jnp.asarray(b2_np).astype(jnp.bfloat16)
    w0y = W0[:COORD_DIM]
    w0x = W0[COORD_DIM:]
    lat2 = jnp.concatenate([latent_tokens_coord, latent_tokens_coord], 0)

    ltab = pl.pallas_call(
        _lift_body,
        grid=(10,),
        in_specs=[pl.BlockSpec((2000, IN_CH), lambda i: (i, 0)),
                  pl.BlockSpec((OUT_CH, IN_CH), lambda i: (0, 0)),
                  pl.BlockSpec((1, OUT_CH), lambda i: (0, 0))],
        out_specs=pl.BlockSpec((2000, OUT_CH), lambda i: (i, 0)),
        out_shape=jax.ShapeDtypeStruct((BATCH * N_NODES, OUT_CH), jnp.float32),
    )(pnd_flat, W_lift, b_lift.reshape(1, OUT_CH))

    s_np = np.zeros((2048, 128), np.float32)
    for n in range(2048):
        s_np[n, n // 16] = 1.0
    s_mat = jnp.asarray(s_np).astype(jnp.bfloat16)
    words, ccnt = pl.pallas_call(
        _mask_body,
        grid=(BATCH, N_LATENT // 256, N_PAD // 2048),
        in_specs=[pl.BlockSpec((256, COORD_DIM), lambda b, qt, nt: (qt, 0)),
                  pl.BlockSpec((1, COORD_DIM, 2048), lambda b, qt, nt: (b, 0, nt)),
                  pl.BlockSpec((2048, 128), lambda b, qt, nt: (0, 0)),
                  pl.BlockSpec((2048, 128), lambda b, qt, nt: (0, 0))],
        out_specs=[pl.BlockSpec((1, 256, 128), lambda b, qt, nt: (b, qt, nt)),
                   pl.BlockSpec((1, 256, 128), lambda b, qt, nt: (b, qt, nt))],
        out_shape=[jax.ShapeDtypeStruct((BATCH, N_LATENT, WORDS), jnp.int32),
                   jax.ShapeDtypeStruct((BATCH, N_LATENT, WORDS), jnp.float32)],
    )(latent_tokens_coord, xT, b2, s_mat)

    tri = jnp.asarray(np.triu(np.ones((WORDS, WORDS), np.float32), 1)
                      ).astype(jnp.bfloat16)
    prefix, counts = pl.pallas_call(
        _prefix_body,
        grid=(BATCH, N_LATENT // 256),
        in_specs=[pl.BlockSpec((1, 256, WORDS), lambda b, qt: (b, qt, 0)),
                  pl.BlockSpec((WORDS, WORDS), lambda b, qt: (0, 0))],
        out_specs=[pl.BlockSpec((1, 256, WORDS), lambda b, qt: (b, qt, 0)),
                   pl.BlockSpec((1, 1, 256), lambda b, qt: (b, 0, qt))],
        out_shape=[jax.ShapeDtypeStruct((BATCH, N_LATENT, WORDS), jnp.int32),
                   jax.ShapeDtypeStruct((BATCH, 1, N_LATENT), jnp.float32)],
    )(ccnt, tri)

    words_flat = words.reshape(NQ, WORDS)
    prefix_flat = prefix.reshape(NQ, WORDS)

    ygx, ygy, ygz, lg = pl.kernel(
        _sc_body,
        out_type=[jax.ShapeDtypeStruct((NQ, KMAX), jnp.float32),
                  jax.ShapeDtypeStruct((NQ, KMAX), jnp.float32),
                  jax.ShapeDtypeStruct((NQ, KMAX), jnp.float32),
                  jax.ShapeDtypeStruct((NQ, KMAX, OUT_CH), jnp.float32)],
        mesh=plsc.VectorSubcoreMesh(core_axis_name="c", subcore_axis_name="s",
                                    num_cores=2, num_subcores=16),
        compiler_params=pltpu.CompilerParams(needs_layout_passes=False),
        scratch_types=[pltpu.VMEM((WORDS,), jnp.int32),
                       pltpu.VMEM((WORDS,), jnp.int32),
                       pltpu.VMEM((KMAX + 16,), jnp.int32),
                       pltpu.VMEM((N_NODES,), jnp.float32),
                       pltpu.VMEM((N_NODES,), jnp.float32),
                       pltpu.VMEM((N_NODES,), jnp.float32),
                       pltpu.VMEM((KMAX,), jnp.float32),
                       pltpu.VMEM((KMAX,), jnp.float32),
                       pltpu.VMEM((KMAX,), jnp.float32),
                       pltpu.VMEM((KMAX, OUT_CH), jnp.float32),
                       pltpu.VMEM_SHARED((N_NODES, OUT_CH), jnp.float32),
                       pltpu.SemaphoreType.DMA],
    )(words_flat, prefix_flat, cx, cy, cz, ltab)

    lg_flat = lg.reshape(NQ * KMAX, OUT_CH)
    counts3 = counts.reshape(NQ // QT, 1, QT)
    out_flat = pl.pallas_call(
        _msg_body,
        grid=(NQ // QT,),
        in_specs=[pl.BlockSpec((QT, KMAX), lambda i: (i, 0)),
                  pl.BlockSpec((QT, KMAX), lambda i: (i, 0)),
                  pl.BlockSpec((QT, KMAX), lambda i: (i, 0)),
                  pl.BlockSpec((QT * KMAX, OUT_CH), lambda i: (i, 0)),
                  pl.BlockSpec((QT, COORD_DIM), lambda i: (i, 0)),
                  pl.BlockSpec((1, 1, QT), lambda i: (i, 0, 0)),
                  pl.BlockSpec((COORD_DIM, HIDDEN), lambda i: (0, 0)),
                  pl.BlockSpec((COORD_DIM, HIDDEN), lambda i: (0, 0)),
                  pl.BlockSpec((1, HIDDEN), lambda i: (0, 0)),
                  pl.BlockSpec((HIDDEN, OUT_CH), lambda i: (0, 0)),
                  pl.BlockSpec((1, OUT_CH), lambda i: (0, 0))],
        out_specs=pl.BlockSpec((QT, OUT_CH), lambda i: (i, 0)),
        out_shape=jax.ShapeDtypeStruct((NQ, OUT_CH), jnp.float32),
    )(ygx, ygy, ygz, lg_flat, lat2, counts3, w0y, w0x, b0.reshape(1, HIDDEN),
      W1, b1.reshape(1, OUT_CH))

    return out_flat.reshape(BATCH, N_LATENT, OUT_CH)


# speedup vs baseline: 1.2084x; 1.2084x over previous
import functools

import jax
import jax.numpy as jnp
import numpy as np
from jax import lax
from jax.experimental import pallas as pl
from jax.experimental.pallas import tpu as pltpu
from jax.experimental.pallas import tpu_sc as plsc

COORD_DIM = 3
IN_CH = 128
OUT_CH = 128
HIDDEN = 64
RADIUS = 0.12
N_NODES = 10000
N_LATENT = 1024
BATCH = 2

N_PAD = 10240
WORDS = N_PAD // 16
WORDS_REAL = N_NODES // 16
KMAX = 128
NQ = BATCH * N_LATENT
NWORKERS = 32
QPW = N_LATENT // 16


def _lift_body(x_ref, w_ref, b_ref, o_ref):
    o_ref[...] = (lax.dot_general(
        x_ref[...], w_ref[...], (((1,), (1,)), ((), ())),
        preferred_element_type=jnp.float32) + b_ref[...])


def _mask_body(q_ref, x_ref, b2_ref, s_ref, w_ref, cc_ref):
    qc = q_ref[...]
    x3 = x_ref[0]
    r2 = RADIUS * RADIUS
    acc = jnp.zeros((256, 2048), jnp.float32)
    for d in range(COORD_DIM):
        diff = qc[:, d:d + 1] - x3[d:d + 1, :]
        acc = acc + diff * diff
    maskb = (acc <= r2).astype(jnp.float32)
    words = lax.dot_general(maskb, b2_ref[...], (((1,), (0,)), ((), ())),
                            preferred_element_type=jnp.float32)
    w_ref[0] = words.astype(jnp.int32)
    cc_ref[0] = lax.dot_general(maskb, s_ref[...], (((1,), (0,)), ((), ())),
                                preferred_element_type=jnp.float32)


def _prefix_body(cc_ref, tri_ref, p_ref, c_ref):
    cc = cc_ref[0]
    pre = lax.dot_general(cc, tri_ref[...],
                          (((1,), (0,)), ((), ())),
                          preferred_element_type=jnp.float32)
    p_ref[0] = jnp.minimum(pre, float(KMAX)).astype(jnp.int32)
    c_ref[...] = jnp.sum(cc, axis=1).reshape(1, 1, 256)


def _sc_body(words_hbm, prefix_hbm, cx_hbm, cy_hbm, cz_hbm, ltab_hbm,
             ygx_hbm, ygy_hbm, ygz_hbm, lg_hbm,
             wbuf, pbuf, idxbuf, wlist, cxb, cyb, czb, yxb, yyb, yzb, lbuf,
             spl, sem0):
    cid = lax.axis_index("c")
    sid = lax.axis_index("s")
    iota = lax.broadcasted_iota(jnp.int32, (16,), 0)
    pltpu.sync_copy(cx_hbm.at[pl.ds(cid * N_NODES, N_NODES)], cxb)
    pltpu.sync_copy(cy_hbm.at[pl.ds(cid * N_NODES, N_NODES)], cyb)
    pltpu.sync_copy(cz_hbm.at[pl.ds(cid * N_NODES, N_NODES)], czb)
    @pl.when(sid == 0)
    def _():
        pltpu.sync_copy(ltab_hbm.at[pl.ds(cid * N_NODES, N_NODES)], spl)
    plsc.subcore_barrier()

    def per_q(i, carry):
        q = cid * N_LATENT + sid * QPW + i
        pltpu.sync_copy(words_hbm.at[q], wbuf)
        pltpu.sync_copy(prefix_hbm.at[q], pbuf)
        for t in range(KMAX // 16):
            idxbuf[pl.ds(t * 16, 16)] = jnp.zeros((16,), jnp.int32)

        for t in range(WORDS // 16 + 1):
            wlist[pl.ds(t * 16, 16)] = jnp.full((16,), WORDS - 1, jnp.int32)

        def lvla(g, nw):
            wv = wbuf[pl.ds(g * 16, 16)]
            m = wv != 0
            plsc.store_compressed(wlist.at[pl.ds(nw, 16)], g * 16 + iota,
                                  mask=m)
            return nw + plsc.all_reduce_population_count(m)[0]

        nw = lax.fori_loop(0, WORDS // 16, lvla, jnp.int32(0))

        def lvlb(u, carry2):
            jv = wlist[pl.ds(u * 16, 16)]
            mvv = plsc.load_gather(wbuf, [jv])
            bases = plsc.load_gather(pbuf, [jv])
            for l in range(16):
                msk = ((jnp.broadcast_to(mvv[l], (16,)) >> iota) & 1) > 0
                nidx = jv[l] * 16 + iota
                plsc.store_compressed(idxbuf.at[pl.ds(bases[l], 16)], nidx,
                                      mask=msk)
            return carry2

        lax.fori_loop(0, (nw + 15) >> 4, lvlb, jnp.int32(0))
        cnt = pbuf[pl.ds(624, 16)][1]
        nch = (cnt + 15) >> 4

        def gch(t, carry3):
            pltpu.async_copy(spl.at[idxbuf.at[pl.ds(t * 16, 16)]],
                             lbuf.at[pl.ds(t * 16, 16)], sem0).wait()
            return carry3

        lax.fori_loop(0, nch, gch, jnp.int32(0))
        for t in range(KMAX // 16):
            iv = idxbuf[pl.ds(t * 16, 16)]
            yxb[pl.ds(t * 16, 16)] = plsc.load_gather(cxb, [iv])
            yyb[pl.ds(t * 16, 16)] = plsc.load_gather(cyb, [iv])
            yzb[pl.ds(t * 16, 16)] = plsc.load_gather(czb, [iv])
        pltpu.sync_copy(lbuf, lg_hbm.at[q])
        pltpu.sync_copy(yxb, ygx_hbm.at[q])
        pltpu.sync_copy(yyb, ygy_hbm.at[q])
        pltpu.sync_copy(yzb, ygz_hbm.at[q])
        return carry

    lax.fori_loop(0, QPW, per_q, jnp.int32(0))


QT = 16


def _msg_body(yx_ref, yy_ref, yz_ref, l_ref, lat_ref, c_ref, w0y_ref,
              w0x_ref, b0_ref, w1_ref, b1_ref, o_ref):
    w0 = w0y_ref[...]
    a = (yx_ref[...][:, :, None] * w0[0][None, None, :]
         + yy_ref[...][:, :, None] * w0[1][None, None, :]
         + yz_ref[...][:, :, None] * w0[2][None, None, :])
    bq = lax.dot_general(lat_ref[...], w0x_ref[...], (((1,), (0,)), ((), ())),
                         preferred_element_type=jnp.float32) + b0_ref[...]
    pre = a + bq[:, None, :]
    g = jax.nn.gelu(pre)
    k = lax.dot_general(g.reshape(QT * KMAX, HIDDEN), w1_ref[...],
                        (((1,), (0,)), ((), ())),
                        preferred_element_type=jnp.float32) + b1_ref[...]
    msg = (k * l_ref[...]).reshape(QT, KMAX, OUT_CH)
    cnt = c_ref[0, 0, :]
    io3 = lax.broadcasted_iota(jnp.int32, (QT, KMAX, OUT_CH), 1)
    valid = io3 < cnt.astype(jnp.int32)[:, None, None]
    s = jnp.sum(jnp.where(valid, msg, 0.0), axis=1)
    denom = jnp.maximum(cnt, 1.0)
    o_ref[...] = s / denom[:, None]


def kernel(x_coord, pndata, latent_tokens_coord, W_lift, b_lift, W0, b0, W1, b1):
    pnd_flat = pndata.reshape(BATCH * N_NODES, IN_CH)
    xT = jnp.transpose(x_coord, (0, 2, 1))
    xT = jnp.pad(xT, ((0, 0), (0, 0), (0, N_PAD - N_NODES)),
                 constant_values=1e4)
    cx = x_coord[..., 0].reshape(BATCH * N_NODES)
    cy = x_coord[..., 1].reshape(BATCH * N_NODES)
    cz = x_coord[..., 2].reshape(BATCH * N_NODES)
    b2_np = np.zeros((2048, 128), np.float32)
    for n in range(2048):
        b2_np[n, n // 16] = float(1 << (n % 16))
    b2 = jnp.asarray(b2_np)
    w0y = W0[:COORD_DIM]
    w0x = W0[COORD_DIM:]
    lat2 = jnp.concatenate([latent_tokens_coord, latent_tokens_coord], 0)

    ltab = pl.pallas_call(
        _lift_body,
        grid=(10,),
        in_specs=[pl.BlockSpec((2000, IN_CH), lambda i: (i, 0)),
                  pl.BlockSpec((OUT_CH, IN_CH), lambda i: (0, 0)),
                  pl.BlockSpec((1, OUT_CH), lambda i: (0, 0))],
        out_specs=pl.BlockSpec((2000, OUT_CH), lambda i: (i, 0)),
        out_shape=jax.ShapeDtypeStruct((BATCH * N_NODES, OUT_CH), jnp.float32),
    )(pnd_flat, W_lift, b_lift.reshape(1, OUT_CH))

    s_np = np.zeros((2048, 128), np.float32)
    for n in range(2048):
        s_np[n, n // 16] = 1.0
    s_mat = jnp.asarray(s_np)
    words, ccnt = pl.pallas_call(
        _mask_body,
        grid=(BATCH, N_LATENT // 256, N_PAD // 2048),
        in_specs=[pl.BlockSpec((256, COORD_DIM), lambda b, qt, nt: (qt, 0)),
                  pl.BlockSpec((1, COORD_DIM, 2048), lambda b, qt, nt: (b, 0, nt)),
                  pl.BlockSpec((2048, 128), lambda b, qt, nt: (0, 0)),
                  pl.BlockSpec((2048, 128), lambda b, qt, nt: (0, 0))],
        out_specs=[pl.BlockSpec((1, 256, 128), lambda b, qt, nt: (b, qt, nt)),
                   pl.BlockSpec((1, 256, 128), lambda b, qt, nt: (b, qt, nt))],
        out_shape=[jax.ShapeDtypeStruct((BATCH, N_LATENT, WORDS), jnp.int32),
                   jax.ShapeDtypeStruct((BATCH, N_LATENT, WORDS), jnp.float32)],
    )(latent_tokens_coord, xT, b2, s_mat)

    tri = jnp.asarray(np.triu(np.ones((WORDS, WORDS), np.float32), 1))
    prefix, counts = pl.pallas_call(
        _prefix_body,
        grid=(BATCH, N_LATENT // 256),
        in_specs=[pl.BlockSpec((1, 256, WORDS), lambda b, qt: (b, qt, 0)),
                  pl.BlockSpec((WORDS, WORDS), lambda b, qt: (0, 0))],
        out_specs=[pl.BlockSpec((1, 256, WORDS), lambda b, qt: (b, qt, 0)),
                   pl.BlockSpec((1, 1, 256), lambda b, qt: (b, 0, qt))],
        out_shape=[jax.ShapeDtypeStruct((BATCH, N_LATENT, WORDS), jnp.int32),
                   jax.ShapeDtypeStruct((BATCH, 1, N_LATENT), jnp.float32)],
    )(ccnt, tri)

    words_flat = words.reshape(NQ, WORDS)
    prefix_flat = prefix.reshape(NQ, WORDS)

    ygx, ygy, ygz, lg = pl.kernel(
        _sc_body,
        out_type=[jax.ShapeDtypeStruct((NQ, KMAX), jnp.float32),
                  jax.ShapeDtypeStruct((NQ, KMAX), jnp.float32),
                  jax.ShapeDtypeStruct((NQ, KMAX), jnp.float32),
                  jax.ShapeDtypeStruct((NQ, KMAX, OUT_CH), jnp.float32)],
        mesh=plsc.VectorSubcoreMesh(core_axis_name="c", subcore_axis_name="s",
                                    num_cores=2, num_subcores=16),
        compiler_params=pltpu.CompilerParams(needs_layout_passes=False),
        scratch_types=[pltpu.VMEM((WORDS,), jnp.int32),
                       pltpu.VMEM((WORDS,), jnp.int32),
                       pltpu.VMEM((KMAX + 16,), jnp.int32),
                       pltpu.VMEM((WORDS + 16,), jnp.int32),
                       pltpu.VMEM((N_NODES,), jnp.float32),
                       pltpu.VMEM((N_NODES,), jnp.float32),
                       pltpu.VMEM((N_NODES,), jnp.float32),
                       pltpu.VMEM((KMAX,), jnp.float32),
                       pltpu.VMEM((KMAX,), jnp.float32),
                       pltpu.VMEM((KMAX,), jnp.float32),
                       pltpu.VMEM((KMAX, OUT_CH), jnp.float32),
                       pltpu.VMEM_SHARED((N_NODES, OUT_CH), jnp.float32),
                       pltpu.SemaphoreType.DMA],
    )(words_flat, prefix_flat, cx, cy, cz, ltab)

    lg_flat = lg.reshape(NQ * KMAX, OUT_CH)
    counts3 = counts.reshape(NQ // QT, 1, QT)
    out_flat = pl.pallas_call(
        _msg_body,
        grid=(NQ // QT,),
        in_specs=[pl.BlockSpec((QT, KMAX), lambda i: (i, 0)),
                  pl.BlockSpec((QT, KMAX), lambda i: (i, 0)),
                  pl.BlockSpec((QT, KMAX), lambda i: (i, 0)),
                  pl.BlockSpec((QT * KMAX, OUT_CH), lambda i: (i, 0)),
                  pl.BlockSpec((QT, COORD_DIM), lambda i: (i, 0)),
                  pl.BlockSpec((1, 1, QT), lambda i: (i, 0, 0)),
                  pl.BlockSpec((COORD_DIM, HIDDEN), lambda i: (0, 0)),
                  pl.BlockSpec((COORD_DIM, HIDDEN), lambda i: (0, 0)),
                  pl.BlockSpec((1, HIDDEN), lambda i: (0, 0)),
                  pl.BlockSpec((HIDDEN, OUT_CH), lambda i: (0, 0)),
                  pl.BlockSpec((1, OUT_CH), lambda i: (0, 0))],
        out_specs=pl.BlockSpec((QT, OUT_CH), lambda i: (i, 0)),
        out_shape=jax.ShapeDtypeStruct((NQ, OUT_CH), jnp.float32),
    )(ygx, ygy, ygz, lg_flat, lat2, counts3, w0y, w0x, b0.reshape(1, HIDDEN),
      W1, b1.reshape(1, OUT_CH))

    return out_flat.reshape(BATCH, N_LATENT, OUT_CH)


# speedup vs baseline: 1.4464x; 1.1970x over previous
import jax
import jax.numpy as jnp
import numpy as np
from jax import lax
from jax.experimental import pallas as pl
from jax.experimental.pallas import tpu as pltpu
from jax.experimental.pallas import tpu_sc as plsc

COORD_DIM = 3
IN_CH = 128
OUT_CH = 128
HIDDEN = 64
RADIUS = 0.12
N_NODES = 10000
N_LATENT = 1024
BATCH = 2

N_PAD = 10240
WORDS = N_PAD // 16
WORDS_REAL = N_NODES // 16
KMAX = 128
NQ = BATCH * N_LATENT
NWORKERS = 32
QPW = N_LATENT // 16


def _lift_body(x_ref, w_ref, b_ref, o_ref):
    o_ref[...] = (lax.dot_general(
        x_ref[...], w_ref[...], (((1,), (1,)), ((), ())),
        preferred_element_type=jnp.float32) + b_ref[...])


def _mask_body(q_ref, x_ref, b2_ref, s_ref, w_ref, cc_ref):
    qc = q_ref[...]
    x3 = x_ref[0]
    r2 = RADIUS * RADIUS
    acc = jnp.zeros((256, 2048), jnp.float32)
    for d in range(COORD_DIM):
        diff = qc[:, d:d + 1] - x3[d:d + 1, :]
        acc = acc + diff * diff
    maskb = (acc <= r2).astype(jnp.float32)
    words = lax.dot_general(maskb, b2_ref[...], (((1,), (0,)), ((), ())),
                            preferred_element_type=jnp.float32)
    w_ref[0] = words.astype(jnp.int32)
    cc_ref[0] = lax.dot_general(maskb, s_ref[...], (((1,), (0,)), ((), ())),
                                preferred_element_type=jnp.float32)


def _prefix_body(cc_ref, tri_ref, p_ref, c_ref):
    cc = cc_ref[0]
    pre = lax.dot_general(cc, tri_ref[...],
                          (((1,), (0,)), ((), ())),
                          preferred_element_type=jnp.float32)
    p_ref[0] = jnp.minimum(pre, float(KMAX)).astype(jnp.int32)
    c_ref[...] = jnp.sum(cc, axis=1).reshape(1, 1, 256)


def _sc_body(words_hbm, prefix_hbm, cx_hbm, cy_hbm, cz_hbm, ltab_hbm,
             ygx_hbm, ygy_hbm, ygz_hbm, lg_hbm,
             wbuf, pbuf, wbuf2, pbuf2, idxbuf, wlist, cxb, cyb, czb,
             yxb, yyb, yzb, lbuf, spl, sem0, sem1):
    cid = lax.axis_index("c")
    sid = lax.axis_index("s")
    iota = lax.broadcasted_iota(jnp.int32, (16,), 0)
    pltpu.sync_copy(cx_hbm.at[pl.ds(cid * N_NODES, N_NODES)], cxb)
    pltpu.sync_copy(cy_hbm.at[pl.ds(cid * N_NODES, N_NODES)], cyb)
    pltpu.sync_copy(cz_hbm.at[pl.ds(cid * N_NODES, N_NODES)], czb)
    @pl.when(sid == 0)
    def _():
        pltpu.sync_copy(ltab_hbm.at[pl.ds(cid * N_NODES, N_NODES)], spl)
    plsc.subcore_barrier()

    qbase = cid * N_LATENT + sid * QPW

    def load_rows(qi, wb, pb, sem):
        pltpu.async_copy(words_hbm.at[qi], wb, sem)
        pltpu.async_copy(prefix_hbm.at[qi], pb, sem)

    def drain_rows(qi, wb, pb, sem):
        pltpu.make_async_copy(words_hbm.at[qi], wb, sem).wait()
        pltpu.make_async_copy(prefix_hbm.at[qi], pb, sem).wait()

    def process(q, wb, pb):
        for t in range(KMAX // 16):
            idxbuf[pl.ds(t * 16, 16)] = jnp.zeros((16,), jnp.int32)
        for t in range(WORDS // 16 + 1):
            wlist[pl.ds(t * 16, 16)] = jnp.full((16,), WORDS - 1, jnp.int32)

        def lvla(g, nw):
            wv = wb[pl.ds(g * 16, 16)]
            m = wv != 0
            plsc.store_compressed(wlist.at[pl.ds(nw, 16)], g * 16 + iota,
                                  mask=m)
            return nw + plsc.all_reduce_population_count(m)[0]

        nw = lax.fori_loop(0, WORDS // 16, lvla, jnp.int32(0))

        def lvlb(u, carry2):
            jv = wlist[pl.ds(u * 16, 16)]
            mvv = plsc.load_gather(wb, [jv])
            bases = plsc.load_gather(pb, [jv])
            for l in range(16):
                msk = ((jnp.broadcast_to(mvv[l], (16,)) >> iota) & 1) > 0
                nidx = jv[l] * 16 + iota
                plsc.store_compressed(idxbuf.at[pl.ds(bases[l], 16)], nidx,
                                      mask=msk)
            return carry2

        lax.fori_loop(0, (nw + 15) >> 4, lvlb, jnp.int32(0))
        cnt = pb[pl.ds(624, 16)][1]
        nch = (cnt + 15) >> 4

        def gch(t, carry3):
            pltpu.async_copy(spl.at[idxbuf.at[pl.ds(t * 16, 16)]],
                             lbuf.at[pl.ds(t * 16, 16)], sem0).wait()
            return carry3

        lax.fori_loop(0, nch, gch, jnp.int32(0))
        for t in range(KMAX // 16):
            iv = idxbuf[pl.ds(t * 16, 16)]
            yxb[pl.ds(t * 16, 16)] = plsc.load_gather(cxb, [iv])
            yyb[pl.ds(t * 16, 16)] = plsc.load_gather(cyb, [iv])
            yzb[pl.ds(t * 16, 16)] = plsc.load_gather(czb, [iv])
        pltpu.sync_copy(lbuf, lg_hbm.at[q])
        pltpu.sync_copy(yxb, ygx_hbm.at[q])
        pltpu.sync_copy(yyb, ygy_hbm.at[q])
        pltpu.sync_copy(yzb, ygz_hbm.at[q])

    load_rows(qbase, wbuf, pbuf, sem0)

    def per_pair(ii, carry):
        q0 = qbase + ii * 2
        q1 = q0 + 1
        q2 = jnp.minimum(q0 + 2, qbase + QPW - 1)
        drain_rows(q0, wbuf, pbuf, sem0)
        load_rows(q1, wbuf2, pbuf2, sem1)
        process(q0, wbuf, pbuf)
        drain_rows(q1, wbuf2, pbuf2, sem1)
        load_rows(q2, wbuf, pbuf, sem0)
        process(q1, wbuf2, pbuf2)
        return carry

    lax.fori_loop(0, QPW // 2, per_pair, jnp.int32(0))
    drain_rows(qbase + QPW - 1, wbuf, pbuf, sem0)


QT = 64


def _msg_body(yx_ref, yy_ref, yz_ref, l_ref, lat_ref, c_ref, w0y_ref,
              w0x_ref, b0_ref, w1_ref, b1_ref, o_ref):
    w0 = w0y_ref[...]
    a = (yx_ref[...][:, :, None] * w0[0][None, None, :]
         + yy_ref[...][:, :, None] * w0[1][None, None, :]
         + yz_ref[...][:, :, None] * w0[2][None, None, :])
    bq = lax.dot_general(lat_ref[...], w0x_ref[...], (((1,), (0,)), ((), ())),
                         preferred_element_type=jnp.float32) + b0_ref[...]
    pre = a + bq[:, None, :]
    g = jax.nn.gelu(pre)
    k = lax.dot_general(g.reshape(QT * KMAX, HIDDEN), w1_ref[...],
                        (((1,), (0,)), ((), ())),
                        preferred_element_type=jnp.float32) + b1_ref[...]
    msg = (k * l_ref[...]).reshape(QT, KMAX, OUT_CH)
    cnt = c_ref[0, 0, :]
    io3 = lax.broadcasted_iota(jnp.int32, (QT, KMAX, OUT_CH), 1)
    valid = io3 < cnt.astype(jnp.int32)[:, None, None]
    s = jnp.sum(jnp.where(valid, msg, 0.0), axis=1)
    denom = jnp.maximum(cnt, 1.0)
    o_ref[...] = s / denom[:, None]


def kernel(x_coord, pndata, latent_tokens_coord, W_lift, b_lift, W0, b0, W1, b1):
    pnd_flat = pndata.reshape(BATCH * N_NODES, IN_CH)
    xT = jnp.transpose(x_coord, (0, 2, 1))
    xT = jnp.pad(xT, ((0, 0), (0, 0), (0, N_PAD - N_NODES)),
                 constant_values=1e4)
    cx = x_coord[..., 0].reshape(BATCH * N_NODES)
    cy = x_coord[..., 1].reshape(BATCH * N_NODES)
    cz = x_coord[..., 2].reshape(BATCH * N_NODES)
    b2_np = np.zeros((2048, 128), np.float32)
    for n in range(2048):
        b2_np[n, n // 16] = float(1 << (n % 16))
    b2 = jnp.asarray(b2_np)
    w0y = W0[:COORD_DIM]
    w0x = W0[COORD_DIM:]
    lat2 = jnp.concatenate([latent_tokens_coord, latent_tokens_coord], 0)

    ltab = pl.pallas_call(
        _lift_body,
        grid=(10,),
        in_specs=[pl.BlockSpec((2000, IN_CH), lambda i: (i, 0)),
                  pl.BlockSpec((OUT_CH, IN_CH), lambda i: (0, 0)),
                  pl.BlockSpec((1, OUT_CH), lambda i: (0, 0))],
        out_specs=pl.BlockSpec((2000, OUT_CH), lambda i: (i, 0)),
        out_shape=jax.ShapeDtypeStruct((BATCH * N_NODES, OUT_CH), jnp.float32),
    )(pnd_flat, W_lift, b_lift.reshape(1, OUT_CH))

    s_np = np.zeros((2048, 128), np.float32)
    for n in range(2048):
        s_np[n, n // 16] = 1.0
    s_mat = jnp.asarray(s_np)
    words, ccnt = pl.pallas_call(
        _mask_body,
        grid=(BATCH, N_LATENT // 256, N_PAD // 2048),
        in_specs=[pl.BlockSpec((256, COORD_DIM), lambda b, qt, nt: (qt, 0)),
                  pl.BlockSpec((1, COORD_DIM, 2048), lambda b, qt, nt: (b, 0, nt)),
                  pl.BlockSpec((2048, 128), lambda b, qt, nt: (0, 0)),
                  pl.BlockSpec((2048, 128), lambda b, qt, nt: (0, 0))],
        out_specs=[pl.BlockSpec((1, 256, 128), lambda b, qt, nt: (b, qt, nt)),
                   pl.BlockSpec((1, 256, 128), lambda b, qt, nt: (b, qt, nt))],
        out_shape=[jax.ShapeDtypeStruct((BATCH, N_LATENT, WORDS), jnp.int32),
                   jax.ShapeDtypeStruct((BATCH, N_LATENT, WORDS), jnp.float32)],
    )(latent_tokens_coord, xT, b2, s_mat)

    tri = jnp.asarray(np.triu(np.ones((WORDS, WORDS), np.float32), 1))
    prefix, counts = pl.pallas_call(
        _prefix_body,
        grid=(BATCH, N_LATENT // 256),
        in_specs=[pl.BlockSpec((1, 256, WORDS), lambda b, qt: (b, qt, 0)),
                  pl.BlockSpec((WORDS, WORDS), lambda b, qt: (0, 0))],
        out_specs=[pl.BlockSpec((1, 256, WORDS), lambda b, qt: (b, qt, 0)),
                   pl.BlockSpec((1, 1, 256), lambda b, qt: (b, 0, qt))],
        out_shape=[jax.ShapeDtypeStruct((BATCH, N_LATENT, WORDS), jnp.int32),
                   jax.ShapeDtypeStruct((BATCH, 1, N_LATENT), jnp.float32)],
    )(ccnt, tri)

    words_flat = words.reshape(NQ, WORDS)
    prefix_flat = prefix.reshape(NQ, WORDS)

    ygx, ygy, ygz, lg = pl.kernel(
        _sc_body,
        out_type=[jax.ShapeDtypeStruct((NQ, KMAX), jnp.float32),
                  jax.ShapeDtypeStruct((NQ, KMAX), jnp.float32),
                  jax.ShapeDtypeStruct((NQ, KMAX), jnp.float32),
                  jax.ShapeDtypeStruct((NQ, KMAX, OUT_CH), jnp.float32)],
        mesh=plsc.VectorSubcoreMesh(core_axis_name="c", subcore_axis_name="s",
                                    num_cores=2, num_subcores=16),
        compiler_params=pltpu.CompilerParams(needs_layout_passes=False),
        scratch_types=[pltpu.VMEM((WORDS,), jnp.int32),
                       pltpu.VMEM((WORDS,), jnp.int32),
                       pltpu.VMEM((WORDS,), jnp.int32),
                       pltpu.VMEM((WORDS,), jnp.int32),
                       pltpu.VMEM((KMAX + 16,), jnp.int32),
                       pltpu.VMEM((WORDS + 16,), jnp.int32),
                       pltpu.VMEM((N_NODES,), jnp.float32),
                       pltpu.VMEM((N_NODES,), jnp.float32),
                       pltpu.VMEM((N_NODES,), jnp.float32),
                       pltpu.VMEM((KMAX,), jnp.float32),
                       pltpu.VMEM((KMAX,), jnp.float32),
                       pltpu.VMEM((KMAX,), jnp.float32),
                       pltpu.VMEM((KMAX, OUT_CH), jnp.float32),
                       pltpu.VMEM_SHARED((N_NODES, OUT_CH), jnp.float32),
                       pltpu.SemaphoreType.DMA,
                       pltpu.SemaphoreType.DMA],
    )(words_flat, prefix_flat, cx, cy, cz, ltab)

    lg_flat = lg.reshape(NQ * KMAX, OUT_CH)
    counts3 = counts.reshape(NQ // QT, 1, QT)
    out_flat = pl.pallas_call(
        _msg_body,
        grid=(NQ // QT,),
        in_specs=[pl.BlockSpec((QT, KMAX), lambda i: (i, 0)),
                  pl.BlockSpec((QT, KMAX), lambda i: (i, 0)),
                  pl.BlockSpec((QT, KMAX), lambda i: (i, 0)),
                  pl.BlockSpec((QT * KMAX, OUT_CH), lambda i: (i, 0)),
                  pl.BlockSpec((QT, COORD_DIM), lambda i: (i, 0)),
                  pl.BlockSpec((1, 1, QT), lambda i: (i, 0, 0)),
                  pl.BlockSpec((COORD_DIM, HIDDEN), lambda i: (0, 0)),
                  pl.BlockSpec((COORD_DIM, HIDDEN), lambda i: (0, 0)),
                  pl.BlockSpec((1, HIDDEN), lambda i: (0, 0)),
                  pl.BlockSpec((HIDDEN, OUT_CH), lambda i: (0, 0)),
                  pl.BlockSpec((1, OUT_CH), lambda i: (0, 0))],
        out_specs=pl.BlockSpec((QT, OUT_CH), lambda i: (i, 0)),
        out_shape=jax.ShapeDtypeStruct((NQ, OUT_CH), jnp.float32),
    )(ygx, ygy, ygz, lg_flat, lat2, counts3, w0y, w0x, b0.reshape(1, HIDDEN),
      W1, b1.reshape(1, OUT_CH))

    return out_flat.reshape(BATCH, N_LATENT, OUT_CH)
